# Initial kernel scaffold; baseline (speedup 1.0000x reference)
import jax
import jax.numpy as jnp
from jax.experimental import pallas as pl


def kernel(x, weather_data, is_solar, unit_ids, c_prime, alpha, alpha_prime, ssrd_scale, A, eta):
    raise NotImplementedError("write your pallas kernel here")


# speedup vs baseline: 1.9304x; 1.9304x over previous
import functools

import jax
import jax.numpy as jnp
from jax.experimental import pallas as pl
from jax.experimental.pallas import tpu as pltpu

B, S, D = 64, 4096, 128
BLK = 1024
P_MAX = 500.0
P_MIN = 0.0


def _body(params_ref, solar_ref, x_ref, w_ref, o_ref):
    b = pl.program_id(0)
    xv = x_ref[0]
    sol = solar_ref[b, 0]

    @pl.when(sol == 1)
    def _():
        coef = params_ref[0, 0]
        scale = params_ref[0, 1]
        w = w_ref[0]
        f = coef * jnp.clip(w * scale, 0.01, 1.0)
        a = xv * f
        t = jnp.sum(a, axis=1, keepdims=True)
        mx = jnp.max(a, axis=1, keepdims=True)
        mn = jnp.min(a, axis=1, keepdims=True)
        rng = jnp.maximum(mx - mn, 1.0)
        lmin, lmax = -rng, rng
        for _ in range(5):
            mid = 0.5 * (lmin + lmax)
            tot = jnp.sum(jnp.clip(a - mid, P_MIN, P_MAX), axis=1, keepdims=True)
            conv = jnp.abs(tot - t) < 0.1
            lmin = jnp.where((tot > t) & (~conv), mid, lmin)
            lmax = jnp.where((tot <= t) & (~conv), mid, lmax)
        lam = 0.5 * (lmin + lmax)
        o_ref[0] = jnp.clip(a - lam, P_MIN, P_MAX)

    @pl.when(sol != 1)
    def _():
        o_ref[0] = jnp.maximum(xv, 0.0)


@jax.jit
def _run(x, w3, solar, params):
    grid = (B, S // BLK)
    return pl.pallas_call(
        _body,
        grid=grid,
        in_specs=[
            pl.BlockSpec(memory_space=pltpu.SMEM),
            pl.BlockSpec(memory_space=pltpu.SMEM),
            pl.BlockSpec((1, BLK, D), lambda b, s: (b, s, 0)),
            pl.BlockSpec((1, BLK, 1), lambda b, s: (b, s, 0)),
        ],
        out_specs=pl.BlockSpec((1, BLK, D), lambda b, s: (b, s, 0)),
        out_shape=jax.ShapeDtypeStruct((B, S, D), jnp.float32),
        compiler_params=pltpu.CompilerParams(
            dimension_semantics=("parallel", "parallel"),
        ),
    )(params, solar, x, w3)


def kernel(x, weather_data, is_solar, unit_ids, c_prime, alpha, alpha_prime,
           ssrd_scale, A, eta):
    coef = c_prime * A * eta / (alpha + alpha_prime) * P_MAX
    params = jnp.stack([coef, ssrd_scale]).reshape(1, 2).astype(jnp.float32)
    w3 = weather_data.reshape(B, S, 1)
    return _run(x, w3, is_solar, params)
